# Initial kernel scaffold; baseline (speedup 1.0000x reference)
import jax
import jax.numpy as jnp
from jax.experimental import pallas as pl


def kernel(x, adj_edge_index, adj_edge_weight, weight):
    raise NotImplementedError("write your pallas kernel here")


# speedup vs baseline: 4.5608x; 4.5608x over previous
import functools

import jax
import jax.numpy as jnp
from jax import lax
from jax.experimental import pallas as pl
from jax.experimental.pallas import tpu as pltpu
from jax.experimental.pallas import tpu_sc as plsc

N_NODES = 10000
N_EDGES = 320000
D = 128

NC = 2
NS = 16
NW = NC * NS
EDGES_PER_W = N_EDGES // NW
CHUNK = 80
NCHUNK = EDGES_PER_W // CHUNK
N_PAD = 10240
ROWS_PER_TILE = N_PAD // NS


def _spmm_sc(x, row, col, ew):
  mesh = plsc.VectorSubcoreMesh(core_axis_name="c", subcore_axis_name="s")

  @functools.partial(
      pl.kernel,
      mesh=mesh,
      out_type=jax.ShapeDtypeStruct((NC, N_PAD, D), jnp.float32),
      scratch_types=[
          pltpu.VMEM((CHUNK,), jnp.int32),
          pltpu.VMEM((CHUNK,), jnp.int32),
          pltpu.VMEM((CHUNK,), jnp.float32),
          pltpu.VMEM((CHUNK, D), jnp.float32),
          pltpu.VMEM((128, D), jnp.float32),
          pltpu.VMEM_SHARED((N_PAD, D), jnp.float32),
          pltpu.SemaphoreType.DMA,
      ],
  )
  def k(x_hbm, row_hbm, col_hbm, w_hbm, out_hbm,
        colv, rowv, wv, rows, zbuf, acc, sem):
    i32 = jnp.int32
    c = lax.axis_index("c").astype(i32)
    s = lax.axis_index("s").astype(i32)
    wid = c * i32(NS) + s

    zv = jnp.zeros((16,), jnp.float32)
    def zero_zbuf(i, carry):
      for j in range(D // 16):
        zbuf[i, pl.ds(j * 16, 16)] = zv
      return carry
    lax.fori_loop(jnp.int32(0), jnp.int32(128), zero_zbuf, jnp.int32(0))
    rbase = s * i32(ROWS_PER_TILE)
    for b in range(ROWS_PER_TILE // 128):
      pltpu.sync_copy(zbuf, acc.at[pl.ds(rbase + i32(b * 128), 128)])
    plsc.subcore_barrier()

    ebase = wid * i32(EDGES_PER_W)
    def chunk_body(it, carry):
      base = ebase + it * i32(CHUNK)
      pltpu.sync_copy(col_hbm.at[pl.ds(base, CHUNK)], colv)
      pltpu.sync_copy(w_hbm.at[pl.ds(base, CHUNK)], wv)
      pltpu.sync_copy(row_hbm.at[pl.ds(base, CHUNK)], rowv)
      pltpu.async_copy(x_hbm.at[colv], rows, sem).wait()
      def scale(g, carry2):
        wv16 = wv[pl.ds(g * i32(16), 16)]
        for l in range(16):
          sv = wv16[l]
          e = g * i32(16) + i32(l)
          for j in range(D // 16):
            sl = pl.ds(j * 16, 16)
            rows[e, sl] = rows[e, sl] * sv
        return carry2
      lax.fori_loop(i32(0), i32(CHUNK // 16), scale, jnp.int32(0))
      pltpu.sync_copy(rows, acc.at[rowv], add=True)
      return carry
    lax.fori_loop(i32(0), i32(NCHUNK), chunk_body, jnp.int32(0))

    plsc.subcore_barrier()
    pltpu.sync_copy(acc.at[pl.ds(rbase, ROWS_PER_TILE)],
                    out_hbm.at[c, pl.ds(rbase, ROWS_PER_TILE)])

  return k(x, row, col, ew)


BLK = 400


def _mm_body(p0_ref, p1_ref, w_ref, o_ref):
  agg = p0_ref[...] + p1_ref[...]
  o_ref[...] = jnp.maximum(
      jnp.dot(agg, w_ref[...], preferred_element_type=jnp.float32), 0.0)


def _matmul_tc(p0, p1, weight):
  return pl.pallas_call(
      _mm_body,
      grid=(N_NODES // BLK,),
      in_specs=[
          pl.BlockSpec((BLK, D), lambda i: (i, jnp.int32(0))),
          pl.BlockSpec((BLK, D), lambda i: (i, jnp.int32(0))),
          pl.BlockSpec((D, D), lambda i: (jnp.int32(0), jnp.int32(0))),
      ],
      out_specs=pl.BlockSpec((BLK, D), lambda i: (i, jnp.int32(0))),
      out_shape=jax.ShapeDtypeStruct((N_NODES, D), jnp.float32),
  )(p0, p1, weight)


def kernel(x, adj_edge_index, adj_edge_weight, weight):
  row = adj_edge_index[0].astype(jnp.int32)
  col = adj_edge_index[1].astype(jnp.int32)
  x = x.astype(jnp.float32)
  ew = adj_edge_weight.astype(jnp.float32)
  partials = _spmm_sc(x, row, col, ew)
  return _matmul_tc(partials[0], partials[1], weight.astype(jnp.float32))
